# baseline (speedup 1.0000x reference)
import functools

import jax
import jax.numpy as jnp
from jax import lax
from jax.experimental import pallas as pl
from jax.experimental.pallas import tpu as pltpu
from jax.experimental.pallas import tpu_sc as plsc

_NC = 2
_NS = 16
_NW = _NC * _NS
_IPW = 512
_IT = _IPW // 128


def _gather_body(nj, d, table_hbm, idx_hbm, out_hbm,
                 idx_v, rows0, rows1, pb0, pb1, gsem0, gsem1, ssem0, ssem1):
    wid = lax.axis_index("s") * _NC + lax.axis_index("c")
    ncg = d // 8
    rows = (rows0, rows1)
    pbuf = (pb0, pb1)
    gsem = (gsem0, gsem1)
    ssem = (ssem0, ssem1)
    pltpu.sync_copy(idx_hbm.at[wid], idx_v)
    lanes = lax.iota(jnp.int32, 16)
    pltpu.async_copy(table_hbm.at[idx_v.at[0]], rows[0], gsem[0])

    def pair(t, carry):
        for b in range(2):
            j = 2 * t + b
            @pl.when(j + 1 < nj)
            def _():
                pltpu.async_copy(
                    table_hbm.at[idx_v.at[j + 1]], rows[1 - b], gsem[1 - b])
            pltpu.make_async_copy(
                table_hbm.at[idx_v.at[0]], rows[b], gsem[b]).wait()
            @pl.when(j >= 2)
            def _():
                for cg in range(ncg):
                    pltpu.make_async_copy(
                        pbuf[b].at[0], out_hbm.at[0, 0, pl.ds(0, _IT * 1024)],
                        ssem[b]).wait()

            def trans(it, carry2):
                rbase = it * 128
                obase = it * 1024
                for cg in range(ncg):
                    for s in range(8):
                        col = jnp.full((16,), cg * 8 + s, jnp.int32)
                        for l0 in range(0, 128, 16):
                            vals = plsc.load_gather(
                                rows[b], [rbase + l0 + lanes, col])
                            pbuf[b][cg, pl.ds(obase + s * 128 + l0, 16)] = vals
                return carry2

            lax.fori_loop(0, _IT, trans, 0)
            for cg in range(ncg):
                pltpu.async_copy(
                    pbuf[b].at[cg],
                    out_hbm.at[j, cg, pl.ds(wid * _IT * 1024, _IT * 1024)],
                    ssem[b])
        return carry

    lax.fori_loop(0, nj // 2, pair, 0)
    for b in range(2):
        for cg in range(ncg):
            pltpu.make_async_copy(
                pbuf[b].at[0], out_hbm.at[0, 0, pl.ds(0, _IT * 1024)],
                ssem[b]).wait()


@functools.partial(jax.jit, static_argnames=("nj", "d"))
def _gather(table, idx, nj, d):
    mesh = plsc.VectorSubcoreMesh(core_axis_name="c", subcore_axis_name="s")
    ncg = d // 8
    kfn = pl.kernel(
        functools.partial(_gather_body, nj, d),
        out_type=jax.ShapeDtypeStruct((nj, ncg, _NW * _IT * 1024), table.dtype),
        mesh=mesh,
        scratch_types=[
            pltpu.VMEM((nj, _IPW), jnp.int32),
            pltpu.VMEM((_IPW, d), table.dtype),
            pltpu.VMEM((_IPW, d), table.dtype),
            pltpu.VMEM((ncg, _IT * 1024), table.dtype),
            pltpu.VMEM((ncg, _IT * 1024), table.dtype),
            pltpu.SemaphoreType.DMA,
            pltpu.SemaphoreType.DMA,
            pltpu.SemaphoreType.DMA,
            pltpu.SemaphoreType.DMA,
        ],
        compiler_params=pltpu.CompilerParams(
            use_tc_tiling_on_sc=False, needs_layout_passes=False),
    )
    return kfn(table, idx)


def kernel(weight_tensor, index_tensor):
    b0, b1 = index_tensor.shape
    v, d = weight_tensor.shape
    assert b0 % (_NW * 128) == 0 and d % 8 == 0 and 128 % d == 0
    pad = 128 // d
    wp = jnp.pad(weight_tensor, ((0, 0), (0, 128 - d))).reshape(v * pad, d)
    idx = (index_tensor.astype(jnp.int32)
           .reshape(_NW, _IPW, b1).transpose(0, 2, 1)) * pad
    p = _gather(wp, idx, b1, d)
    ncg = d // 8
    return (p.reshape(b1, ncg, _NW * _IT, 8, 128)
            .transpose(2, 4, 0, 1, 3)
            .reshape(b0, b1, d))
